# baseline (speedup 1.0000x reference)
import jax
import jax.numpy as jnp
from jax import lax
from jax.experimental import pallas as pl
from jax.experimental.pallas import tpu as pltpu
from jax.experimental.pallas import tpu_sc as plsc

B, S, D = 4, 8192, 768
F = D // 4
CAP = S // 2
TSR = 4096
TSA = 4096
NTOK = B * S
NGR = NTOK // TSR
NGA = NTOK // TSA
NKV = S // 16


def _sortkey(f32vec):
    i = lax.bitcast_convert_type(f32vec, jnp.int32)
    return i ^ ((i >> 31) & jnp.int32(0x7FFFFFFF))


def _router_body(x_ref, w1_ref, b1_ref, w2p_ref, b2_ref, s_ref, rw_ref):
    xb = x_ref[...]
    h = jnp.dot(xb, w1_ref[...], preferred_element_type=jnp.float32)
    h = jnp.maximum(h + b1_ref[0][None, :], 0.0)
    sm = jnp.dot(h, w2p_ref[...], preferred_element_type=jnp.float32)
    s = sm[:, 0:1] + b2_ref[0, 0]
    s_ref[...] = s
    rw_ref[...] = jax.nn.sigmoid(s)


def _scalar(x):
    return jnp.max(x) if getattr(x, "ndim", 0) else x


def _scan_desc(hist_ref, nvregs, k_target):
    lanes = lax.iota(jnp.int32, 16)

    def body(i, carry):
        done, total, bin_t, above = carry
        v = nvregs - 1 - i
        h = hist_ref[pl.ds(v * 16, 16)]
        s = jnp.sum(h)
        hr = lax.rev(h, dimensions=(0,))
        cum = plsc.cumsum(hr)
        cond = (total + cum) >= k_target
        j = _scalar(plsc.all_reduce_ffs(cond))
        above_here = total + jnp.sum(jnp.where(lanes < j, hr, 0))
        bin_here = v * 16 + 15 - j
        found = jnp.logical_and(done == 0, (total + s) >= k_target)
        return (jnp.where(found, 1, done),
                total + s,
                jnp.where(found, bin_here, bin_t),
                jnp.where(found, above_here, above))

    init = (jnp.int32(0), jnp.int32(0), jnp.int32(0), jnp.int32(0))
    _, _, bin_t, above = lax.fori_loop(0, nvregs, body, init)
    return bin_t, above


def _hist_pass(fbuf_ref, hist_ref, nzero, digit_fn, mask_fn):
    def zbody(i, _):
        hist_ref[pl.ds(i * 16, 16)] = jnp.zeros((16,), jnp.int32)
        return 0

    lax.fori_loop(0, nzero, zbody, 0)
    ones = jnp.ones((16,), jnp.int32)

    def body(i, _):
        kv = _sortkey(fbuf_ref[pl.ds(i * 16, 16)])
        plsc.addupdate_scatter(hist_ref, [digit_fn(kv)], ones,
                               mask=mask_fn(kv))
        return 0

    lax.fori_loop(0, NKV, body, 0)


def _sc_thresh_body(scores_hbm, out_hbm, fbuf, hist, obuf):
    wid = lax.axis_index("s") * 2 + lax.axis_index("c")

    @pl.when(wid < B)
    def _():
        b = wid
        pltpu.sync_copy(scores_hbm.at[b], fbuf)

        true16 = jnp.ones((16,), jnp.bool_)

        _hist_pass(fbuf, hist, 128,
                   lambda kv: (kv >> 21) + 1024,
                   lambda kv: true16)
        t1, above1 = _scan_desc(hist, 128, CAP)
        d1 = t1 - 1024
        k1 = CAP - above1

        _hist_pass(fbuf, hist, 16,
                   lambda kv: (kv >> 13) & 255,
                   lambda kv: (kv >> 21) == d1)
        t2, above2 = _scan_desc(hist, 16, k1)
        p2 = (d1 << 8) | t2
        k2 = k1 - above2

        _hist_pass(fbuf, hist, 16,
                   lambda kv: (kv >> 5) & 255,
                   lambda kv: (kv >> 13) == p2)
        t3, above3 = _scan_desc(hist, 16, k2)
        p3 = (p2 << 8) | t3
        k3 = k2 - above3

        _hist_pass(fbuf, hist, 2,
                   lambda kv: kv & 31,
                   lambda kv: (kv >> 5) == p3)
        t4, _ = _scan_desc(hist, 2, k3)

        vkey = (p3 << 5) | t4
        obuf[...] = jnp.full((16,), vkey, jnp.int32)
        pltpu.sync_copy(obuf, out_hbm.at[b])


def _apply_body(vk_ref, x_ref, s_ref, g_ref, bt_ref, o_ref):
    t = pl.program_id(0)
    b = t // (S // TSA)
    xb = x_ref[...]
    keys = _sortkey(s_ref[...])
    mask = keys >= vk_ref[b]
    o_ref[...] = jnp.where(
        mask, xb * g_ref[0][None, :] + bt_ref[0][None, :], xb)


@jax.jit
def kernel(x, W1, b1, W2, b2, gamma, beta):
    x2 = x.reshape(NTOK, D)
    w2p = jnp.pad(W2, ((0, 0), (0, 127)))

    scores, rw = pl.pallas_call(
        _router_body,
        grid=(NGR,),
        in_specs=[
            pl.BlockSpec((TSR, D), lambda t: (t, 0)),
            pl.BlockSpec((D, F), lambda t: (0, 0)),
            pl.BlockSpec((1, F), lambda t: (0, 0)),
            pl.BlockSpec((F, 128), lambda t: (0, 0)),
            pl.BlockSpec((1, 1), lambda t: (0, 0), memory_space=pltpu.SMEM),
        ],
        out_specs=[
            pl.BlockSpec((TSR, 1), lambda t: (t, 0)),
            pl.BlockSpec((TSR, 1), lambda t: (t, 0)),
        ],
        out_shape=[
            jax.ShapeDtypeStruct((NTOK, 1), jnp.float32),
            jax.ShapeDtypeStruct((NTOK, 1), jnp.float32),
        ],
    )(x2, W1, b1.reshape(1, F), w2p, b2.reshape(1, 1))

    sc_thresh = pl.kernel(
        _sc_thresh_body,
        out_type=jax.ShapeDtypeStruct((B, 16), jnp.int32),
        mesh=plsc.VectorSubcoreMesh(core_axis_name="c", subcore_axis_name="s"),
        compiler_params=pltpu.CompilerParams(needs_layout_passes=False),
        scratch_types=[
            pltpu.VMEM((S,), jnp.float32),
            pltpu.VMEM((2048,), jnp.int32),
            pltpu.VMEM((16,), jnp.int32),
        ],
    )
    vkeys = sc_thresh(scores.reshape(B, S))[:, 0]

    out = pl.pallas_call(
        _apply_body,
        grid=(NGA,),
        in_specs=[
            pl.BlockSpec(memory_space=pltpu.SMEM),
            pl.BlockSpec((TSA, D), lambda t: (t, 0)),
            pl.BlockSpec((TSA, 1), lambda t: (t, 0)),
            pl.BlockSpec((1, D), lambda t: (0, 0)),
            pl.BlockSpec((1, D), lambda t: (0, 0)),
        ],
        out_specs=pl.BlockSpec((TSA, D), lambda t: (t, 0)),
        out_shape=jax.ShapeDtypeStruct((NTOK, D), jnp.float32),
    )(vkeys, x2, scores, gamma.reshape(1, D), beta.reshape(1, D))

    return out.reshape(B, S, D), rw.reshape(B, S, 1)


# speedup vs baseline: 1.0671x; 1.0671x over previous
import jax
import jax.numpy as jnp
from jax import lax
from jax.experimental import pallas as pl
from jax.experimental.pallas import tpu as pltpu
from jax.experimental.pallas import tpu_sc as plsc

B, S, D = 4, 8192, 768
F = D // 4
CAP = S // 2
TSR = 4096
TSA = 4096
NTOK = B * S
NGR = NTOK // TSR
NGA = NTOK // TSA
NKV = S // 16


def _sortkey(f32vec):
    i = lax.bitcast_convert_type(f32vec, jnp.int32)
    return i ^ ((i >> 31) & jnp.int32(0x7FFFFFFF))


def _router_body(x_ref, w1_ref, b1_ref, w2p_ref, b2_ref, s_ref, rw_ref,
                 sf_ref):
    xb = x_ref[...]
    h = jnp.dot(xb, w1_ref[...], preferred_element_type=jnp.float32)
    h = jnp.maximum(h + b1_ref[0][None, :], 0.0)
    sm = jnp.dot(h, w2p_ref[...], preferred_element_type=jnp.float32)
    s = sm[:, 0:1] + b2_ref[0, 0]
    s_ref[...] = s
    rw_ref[...] = jax.nn.sigmoid(s)
    sf_ref[...] = sm[:, 0] + b2_ref[0, 0]


def _scalar(x):
    return jnp.max(x) if getattr(x, "ndim", 0) else x


def _scan_desc(hist_ref, nvregs, k_target, vreg_base=0):
    lanes = lax.iota(jnp.int32, 16)

    def body(i, carry):
        done, total, bin_t, above = carry
        v = nvregs - 1 - i
        h = hist_ref[pl.ds((vreg_base + v) * 16, 16)]
        s = jnp.sum(h)
        hr = lax.rev(h, dimensions=(0,))
        cum = plsc.cumsum(hr)
        cond = (total + cum) >= k_target
        j = _scalar(plsc.all_reduce_ffs(cond))
        above_here = total + jnp.sum(jnp.where(lanes < j, hr, 0))
        bin_here = v * 16 + 15 - j
        found = jnp.logical_and(done == 0, (total + s) >= k_target)
        return (jnp.where(found, 1, done),
                total + s,
                jnp.where(found, bin_here, bin_t),
                jnp.where(found, above_here, above))

    init = (jnp.int32(0), jnp.int32(0), jnp.int32(0), jnp.int32(0))
    _, _, bin_t, above = lax.fori_loop(0, nvregs, body, init)
    return bin_t, above


def _scan_2048(hist_ref, k_target):
    blk_sums = []
    for blk in range(8):
        acc = jnp.zeros((16,), jnp.int32)
        for v in range(16):
            acc = acc + hist_ref[pl.ds((blk * 16 + v) * 16, 16)]
        blk_sums.append(jnp.sum(acc))
    done = jnp.int32(0)
    total = jnp.int32(0)
    tblk = jnp.int32(0)
    above_blk = jnp.int32(0)
    for blk in range(7, -1, -1):
        found = jnp.logical_and(done == 0, (total + blk_sums[blk]) >= k_target)
        tblk = jnp.where(found, blk, tblk)
        above_blk = jnp.where(found, total, above_blk)
        done = jnp.where(found, 1, done)
        total = total + blk_sums[blk]
    t_loc, above_loc = _scan_desc(hist_ref, 16, k_target - above_blk,
                                  vreg_base=tblk * 16)
    return tblk * 256 + t_loc, above_blk + above_loc


def _zero(hist_ref, nzero):
    z = jnp.zeros((16,), jnp.int32)

    def zbody(i, _):
        for u in range(4):
            hist_ref[pl.ds((i * 4 + u) * 16, 16)] = z
        return 0

    if nzero >= 4:
        lax.fori_loop(0, nzero // 4, zbody, 0)
    else:
        for i in range(nzero):
            hist_ref[pl.ds(i * 16, 16)] = z


def _hist_pass(kbuf_ref, hist_ref, nzero, digit_fn, mask_fn):
    _zero(hist_ref, nzero)
    ones = jnp.ones((16,), jnp.int32)

    def body(i, _):
        for u in range(4):
            kv = kbuf_ref[pl.ds((i * 4 + u) * 16, 16)]
            plsc.addupdate_scatter(hist_ref, [digit_fn(kv)], ones,
                                   mask=mask_fn(kv))
        return 0

    lax.fori_loop(0, NKV // 4, body, 0)


def _sc_thresh_body(scores_hbm, out_hbm, fbuf, kbuf, hist, obuf):
    wid = lax.axis_index("s") * 2 + lax.axis_index("c")

    @pl.when(wid < B)
    def _():
        b = wid
        pltpu.sync_copy(scores_hbm.at[pl.ds(b * S, S)], fbuf)

        _zero(hist, 128)
        ones = jnp.ones((16,), jnp.int32)

        def p1body(i, _):
            for u in range(4):
                kv = _sortkey(fbuf[pl.ds((i * 4 + u) * 16, 16)])
                kbuf[pl.ds((i * 4 + u) * 16, 16)] = kv
                plsc.addupdate_scatter(hist, [(kv >> 21) + 1024], ones)
            return 0

        lax.fori_loop(0, NKV // 4, p1body, 0)
        t1, above1 = _scan_2048(hist, CAP)
        d1 = t1 - 1024
        k1 = CAP - above1

        _hist_pass(kbuf, hist, 16,
                   lambda kv: (kv >> 13) & 255,
                   lambda kv: (kv >> 21) == d1)
        t2, above2 = _scan_desc(hist, 16, k1)
        p2 = (d1 << 8) | t2
        k2 = k1 - above2

        _hist_pass(kbuf, hist, 16,
                   lambda kv: (kv >> 5) & 255,
                   lambda kv: (kv >> 13) == p2)
        t3, above3 = _scan_desc(hist, 16, k2)
        p3 = (p2 << 8) | t3
        k3 = k2 - above3

        _hist_pass(kbuf, hist, 2,
                   lambda kv: kv & 31,
                   lambda kv: (kv >> 5) == p3)
        t4, _ = _scan_desc(hist, 2, k3)

        vkey = (p3 << 5) | t4
        obuf[...] = jnp.full((16,), vkey, jnp.int32)
        pltpu.sync_copy(obuf, out_hbm.at[b])


def _apply_body(vk_ref, x_ref, s_ref, g_ref, bt_ref, o_ref):
    t = pl.program_id(0)
    b = t // (S // TSA)
    xb = x_ref[...]
    keys = _sortkey(s_ref[...])
    mask = keys >= vk_ref[b, 0]
    o_ref[...] = jnp.where(
        mask, xb * g_ref[0][None, :] + bt_ref[0][None, :], xb)


@jax.jit
def kernel(x, W1, b1, W2, b2, gamma, beta):
    x2 = x.reshape(NTOK, D)
    w2p = jnp.pad(W2, ((0, 0), (0, 127)))

    router_outs = pl.pallas_call(
        _router_body,
        grid=(NGR,),
        in_specs=[
            pl.BlockSpec((TSR, D), lambda t: (t, 0)),
            pl.BlockSpec((D, F), lambda t: (0, 0)),
            pl.BlockSpec((1, F), lambda t: (0, 0)),
            pl.BlockSpec((F, 128), lambda t: (0, 0)),
            pl.BlockSpec((1, 1), lambda t: (0, 0), memory_space=pltpu.SMEM),
        ],
        out_specs=[
            pl.BlockSpec((TSR, 1), lambda t: (t, 0)),
            pl.BlockSpec((TSR, 1), lambda t: (t, 0)),
            pl.BlockSpec((TSR,), lambda t: (t,)),
        ],
        out_shape=[
            jax.ShapeDtypeStruct((NTOK, 1), jnp.float32),
            jax.ShapeDtypeStruct((NTOK, 1), jnp.float32),
            jax.ShapeDtypeStruct((NTOK,), jnp.float32),
        ],
    )(x2, W1, b1.reshape(1, F), w2p, b2.reshape(1, 1))
    scores, rw, scores_flat = router_outs

    sc_thresh = pl.kernel(
        _sc_thresh_body,
        out_type=jax.ShapeDtypeStruct((B, 16), jnp.int32),
        mesh=plsc.VectorSubcoreMesh(core_axis_name="c", subcore_axis_name="s"),
        compiler_params=pltpu.CompilerParams(needs_layout_passes=False),
        scratch_types=[
            pltpu.VMEM((S,), jnp.float32),
            pltpu.VMEM((S,), jnp.int32),
            pltpu.VMEM((2048,), jnp.int32),
            pltpu.VMEM((16,), jnp.int32),
        ],
    )
    vkeys = sc_thresh(scores_flat)

    out = pl.pallas_call(
        _apply_body,
        grid=(NGA,),
        in_specs=[
            pl.BlockSpec(memory_space=pltpu.SMEM),
            pl.BlockSpec((TSA, D), lambda t: (t, 0)),
            pl.BlockSpec((TSA, 1), lambda t: (t, 0)),
            pl.BlockSpec((1, D), lambda t: (0, 0)),
            pl.BlockSpec((1, D), lambda t: (0, 0)),
        ],
        out_specs=pl.BlockSpec((TSA, D), lambda t: (t, 0)),
        out_shape=jax.ShapeDtypeStruct((NTOK, D), jnp.float32),
    )(vkeys, x2, scores, gamma.reshape(1, D), beta.reshape(1, D))

    return out.reshape(B, S, D), rw.reshape(B, S, 1)
